# baseline (speedup 1.0000x reference)
import jax
import jax.numpy as jnp
from jax import lax
from jax.experimental import pallas as pl
from jax.experimental.pallas import tpu as pltpu

_NEG = -3.4028235e38


def _embed_kernel(L):
    def body(x_ref, we_ref, b_ref, len_ref, o_ref):
        rows = x_ref.shape[0]
        E = we_ref.shape[1]
        emb = jnp.dot(x_ref[...], we_ref[...],
                      preferred_element_type=jnp.float32) + b_ref[...]
        t = lax.rem(lax.broadcasted_iota(jnp.int32, (rows, 1), 0), L)
        emb = jnp.where(t < len_ref[...], emb, 0.0)
        o_ref[...] = jnp.swapaxes(emb.reshape(rows // L, L, E), 0, 1)
    return body


def _bigru_kernel(L, Tb, H):
    G = 3 * H
    NCH = 16

    def sig(x):
        return 0.5 * jnp.tanh(0.5 * x) + 0.5

    def cell(xp, h, whh_ref):
        hp = jnp.dot(h.astype(jnp.bfloat16), whh_ref[...],
                     preferred_element_type=jnp.float32)
        r = sig(xp[:, 0:H] + hp[:, 0:H])
        z = sig(xp[:, H:2 * H] + hp[:, H:2 * H])
        n = jnp.tanh(xp[:, 2 * H:3 * H] + r * hp[:, 2 * H:3 * H])
        return (1.0 - z) * n + z * h

    def body(emb_ref, len_ref, tmax_ref, w0_ref, whh0f_ref, whh0b_ref,
             w1_ref, whh1f_ref, whh1b_ref, out_ref,
             xp_ref, of_ref, ob_ref):
        lens = len_ref[...]
        E = emb_ref.shape[-1]
        Tc = L // NCH

        def trow(s):
            return pl.ds(pl.multiple_of(s * Tb, Tb), Tb)

        for c in range(NCH):
            emb_c = emb_ref[pl.ds(c * Tc, Tc)].reshape(Tc * Tb, E)
            xp_ref[pl.ds(c * Tc * Tb, Tc * Tb), :] = jnp.dot(
                emb_c.astype(jnp.bfloat16), w0_ref[...],
                preferred_element_type=jnp.float32).astype(jnp.bfloat16)

        def step0(s, carry):
            h_f, h_b = carry
            sb = L - 1 - s
            h_f = cell(xp_ref[trow(s), 0:G], h_f, whh0f_ref)
            h_b = cell(xp_ref[trow(sb), G:2 * G], h_b, whh0b_ref)
            of_ref[trow(s), :] = jnp.where(lens > s, h_f,
                                           0.0).astype(jnp.bfloat16)
            ob_ref[trow(sb), :] = jnp.where(lens > sb, h_b,
                                            0.0).astype(jnp.bfloat16)
            return (h_f, h_b)

        zeros = jnp.zeros((Tb, H), jnp.float32)
        lax.fori_loop(0, L, step0, (zeros, zeros), unroll=32)

        for c in range(NCH):
            rows = pl.ds(c * Tc * Tb, Tc * Tb)
            lhs = jnp.concatenate([of_ref[rows, :], ob_ref[rows, :]], axis=1)
            xp_ref[rows, :] = jnp.dot(
                lhs, w1_ref[...],
                preferred_element_type=jnp.float32).astype(jnp.bfloat16)

        def step1(s, carry):
            h_f, h_b, a_f, a_b = carry
            sb = L - 1 - s
            h_f = cell(xp_ref[trow(s), 0:G], h_f, whh1f_ref)
            h_b = cell(xp_ref[trow(sb), G:2 * G], h_b, whh1b_ref)
            a_f = jnp.maximum(a_f, jnp.where(lens > s, h_f, _NEG))
            a_b = jnp.maximum(a_b, jnp.where(lens > sb, h_b, _NEG))
            return (h_f, h_b, a_f, a_b)

        negs = jnp.full((Tb, H), _NEG, jnp.float32)
        _, _, a_f, a_b = lax.fori_loop(0, L, step1,
                                       (zeros, zeros, negs, negs), unroll=32)

        short = lens < tmax_ref[...]
        out_ref[:, 0:H] = jnp.where(short, jnp.maximum(a_f, 0.0), a_f)
        out_ref[:, H:2 * H] = jnp.where(short, jnp.maximum(a_b, 0.0), a_b)

    return body


def kernel(text, lengths, embed_wt, embed_b,
           gru_l0_fwd_wih_p, gru_l0_fwd_whh_p,
           gru_l0_bwd_wih_p, gru_l0_bwd_whh_p,
           gru_l1_fwd_wih_p, gru_l1_fwd_whh_p,
           gru_l1_bwd_wih_p, gru_l1_bwd_whh_p,
           gru_l0_fwd_wih_r, gru_l0_fwd_whh_r,
           gru_l0_bwd_wih_r, gru_l0_bwd_whh_r,
           gru_l1_fwd_wih_r, gru_l1_fwd_whh_r,
           gru_l1_bwd_wih_r, gru_l1_bwd_whh_r):
    B, L, V = text.shape
    E = embed_wt.shape[1]
    H = gru_l0_fwd_whh_p.shape[0]
    G = 3 * H

    len_col = lengths.astype(jnp.int32).reshape(B, 1)
    len_row = jnp.repeat(len_col, L, axis=0)
    tmax = jnp.max(lengths).astype(jnp.int32).reshape(1, 1)

    N = B * L
    tm = min(1024, N)
    n_blk = N // tm
    rows_b = tm // L

    emb_t = pl.pallas_call(
        _embed_kernel(L),
        out_shape=jax.ShapeDtypeStruct((L, B, E), jnp.float32),
        grid=(n_blk,),
        in_specs=[
            pl.BlockSpec((tm, V), lambda i: (i, 0)),
            pl.BlockSpec((V, E), lambda i: (0, 0)),
            pl.BlockSpec((1, E), lambda i: (0, 0)),
            pl.BlockSpec((tm, 1), lambda i: (i, 0)),
        ],
        out_specs=pl.BlockSpec((L, rows_b, E), lambda i: (0, i, 0)),
        compiler_params=pltpu.CompilerParams(
            dimension_semantics=("arbitrary",)),
    )(text.reshape(N, V), embed_wt, embed_b, len_row)

    bf = jnp.bfloat16
    w0 = jnp.concatenate([gru_l0_fwd_wih_p, gru_l0_bwd_wih_p],
                         axis=1).astype(bf)
    w1 = jnp.concatenate([gru_l1_fwd_wih_p, gru_l1_bwd_wih_p],
                         axis=1).astype(bf)
    whh0f = gru_l0_fwd_whh_p.astype(bf)
    whh0b = gru_l0_bwd_whh_p.astype(bf)
    whh1f = gru_l1_fwd_whh_p.astype(bf)
    whh1b = gru_l1_bwd_whh_p.astype(bf)

    out = pl.pallas_call(
        _bigru_kernel(L, B, H),
        out_shape=jax.ShapeDtypeStruct((B, 2 * H), jnp.float32),
        grid=(1,),
        in_specs=[
            pl.BlockSpec((L, B, E), lambda b: (0, 0, 0)),
            pl.BlockSpec((B, 1), lambda b: (0, 0)),
            pl.BlockSpec((1, 1), lambda b: (0, 0)),
            pl.BlockSpec((E, 2 * G), lambda b: (0, 0)),
            pl.BlockSpec((H, G), lambda b: (0, 0)),
            pl.BlockSpec((H, G), lambda b: (0, 0)),
            pl.BlockSpec((2 * H, 2 * G), lambda b: (0, 0)),
            pl.BlockSpec((H, G), lambda b: (0, 0)),
            pl.BlockSpec((H, G), lambda b: (0, 0)),
        ],
        out_specs=pl.BlockSpec((B, 2 * H), lambda b: (0, 0)),
        scratch_shapes=[
            pltpu.VMEM((L * B, 2 * G), jnp.bfloat16),
            pltpu.VMEM((L * B, H), jnp.bfloat16),
            pltpu.VMEM((L * B, H), jnp.bfloat16),
        ],
        compiler_params=pltpu.CompilerParams(
            dimension_semantics=("arbitrary",)),
    )(emb_t, len_col, tmax, w0, whh0f, whh0b, w1, whh1f, whh1b)

    return out


# speedup vs baseline: 1.0565x; 1.0565x over previous
import jax
import jax.numpy as jnp
from jax import lax
from jax.experimental import pallas as pl
from jax.experimental.pallas import tpu as pltpu

_NEG = -3.4028235e38


def _fused_kernel(L, B, H, n_blk):
    G = 3 * H
    NCH = 16

    def sig(x):
        return 0.5 * jnp.tanh(0.5 * x) + 0.5

    def cell(xp, h, whh_ref):
        hp = jnp.dot(h.astype(jnp.bfloat16), whh_ref[...],
                     preferred_element_type=jnp.float32)
        r = sig(xp[:, 0:H] + hp[:, 0:H])
        z = sig(xp[:, H:2 * H] + hp[:, H:2 * H])
        n = jnp.tanh(xp[:, 2 * H:3 * H] + r * hp[:, 2 * H:3 * H])
        return (1.0 - z) * n + z * h

    def body(x_ref, lr_ref, we_ref, b_ref, len_ref, tmax_ref,
             w0_ref, whh0f_ref, whh0b_ref, w1_ref, whh1f_ref, whh1b_ref,
             out_ref, emb_ref, xp_ref, of_ref, ob_ref):
        i = pl.program_id(0)
        tm = x_ref.shape[0]
        E = we_ref.shape[1]
        rows_b = tm // L

        emb = jnp.dot(x_ref[...], we_ref[...],
                      preferred_element_type=jnp.float32) + b_ref[...]
        t = lax.rem(lax.broadcasted_iota(jnp.int32, (tm, 1), 0), L)
        emb = jnp.where(t < lr_ref[...], emb, 0.0)
        emb_ref[:, pl.ds(pl.multiple_of(i * rows_b, rows_b), rows_b), :] = (
            jnp.swapaxes(emb.reshape(rows_b, L, E), 0, 1))

        @pl.when(i == n_blk - 1)
        def _():
            lens = len_ref[...]
            Tc = L // NCH

            def trow(s):
                return pl.ds(pl.multiple_of(s * B, B), B)

            for c in range(NCH):
                emb_c = emb_ref[pl.ds(c * Tc, Tc)].reshape(Tc * B, E)
                xp_ref[pl.ds(c * Tc * B, Tc * B), :] = jnp.dot(
                    emb_c.astype(jnp.bfloat16), w0_ref[...],
                    preferred_element_type=jnp.float32).astype(jnp.bfloat16)

            def step0(s, carry):
                h_f, h_b = carry
                sb = L - 1 - s
                h_f = cell(xp_ref[trow(s), 0:G], h_f, whh0f_ref)
                h_b = cell(xp_ref[trow(sb), G:2 * G], h_b, whh0b_ref)
                of_ref[trow(s), :] = jnp.where(lens > s, h_f,
                                               0.0).astype(jnp.bfloat16)
                ob_ref[trow(sb), :] = jnp.where(lens > sb, h_b,
                                                0.0).astype(jnp.bfloat16)
                return (h_f, h_b)

            zeros = jnp.zeros((B, H), jnp.float32)
            lax.fori_loop(0, L, step0, (zeros, zeros), unroll=32)

            for c in range(NCH):
                rows = pl.ds(c * Tc * B, Tc * B)
                lhs = jnp.concatenate([of_ref[rows, :], ob_ref[rows, :]],
                                      axis=1)
                xp_ref[rows, :] = jnp.dot(
                    lhs, w1_ref[...],
                    preferred_element_type=jnp.float32).astype(jnp.bfloat16)

            def step1(s, carry):
                h_f, h_b, a_f, a_b = carry
                sb = L - 1 - s
                h_f = cell(xp_ref[trow(s), 0:G], h_f, whh1f_ref)
                h_b = cell(xp_ref[trow(sb), G:2 * G], h_b, whh1b_ref)
                a_f = jnp.maximum(a_f, jnp.where(lens > s, h_f, _NEG))
                a_b = jnp.maximum(a_b, jnp.where(lens > sb, h_b, _NEG))
                return (h_f, h_b, a_f, a_b)

            negs = jnp.full((B, H), _NEG, jnp.float32)
            _, _, a_f, a_b = lax.fori_loop(0, L, step1,
                                           (zeros, zeros, negs, negs),
                                           unroll=32)

            short = lens < tmax_ref[...]
            out_ref[:, 0:H] = jnp.where(short, jnp.maximum(a_f, 0.0), a_f)
            out_ref[:, H:2 * H] = jnp.where(short,
                                            jnp.maximum(a_b, 0.0), a_b)

    return body


def kernel(text, lengths, embed_wt, embed_b,
           gru_l0_fwd_wih_p, gru_l0_fwd_whh_p,
           gru_l0_bwd_wih_p, gru_l0_bwd_whh_p,
           gru_l1_fwd_wih_p, gru_l1_fwd_whh_p,
           gru_l1_bwd_wih_p, gru_l1_bwd_whh_p,
           gru_l0_fwd_wih_r, gru_l0_fwd_whh_r,
           gru_l0_bwd_wih_r, gru_l0_bwd_whh_r,
           gru_l1_fwd_wih_r, gru_l1_fwd_whh_r,
           gru_l1_bwd_wih_r, gru_l1_bwd_whh_r):
    B, L, V = text.shape
    E = embed_wt.shape[1]
    H = gru_l0_fwd_whh_p.shape[0]
    G = 3 * H

    len_col = lengths.astype(jnp.int32).reshape(B, 1)
    len_row = jnp.repeat(len_col, L, axis=0)
    tmax = jnp.max(lengths).astype(jnp.int32).reshape(1, 1)

    bf = jnp.bfloat16
    w0 = jnp.concatenate([gru_l0_fwd_wih_p, gru_l0_bwd_wih_p],
                         axis=1).astype(bf)
    w1 = jnp.concatenate([gru_l1_fwd_wih_p, gru_l1_bwd_wih_p],
                         axis=1).astype(bf)
    whh0f = gru_l0_fwd_whh_p.astype(bf)
    whh0b = gru_l0_bwd_whh_p.astype(bf)
    whh1f = gru_l1_fwd_whh_p.astype(bf)
    whh1b = gru_l1_bwd_whh_p.astype(bf)

    N = B * L
    tm = min(512, N)
    n_blk = N // tm

    out = pl.pallas_call(
        _fused_kernel(L, B, H, n_blk),
        out_shape=jax.ShapeDtypeStruct((B, 2 * H), jnp.float32),
        grid=(n_blk,),
        in_specs=[
            pl.BlockSpec((tm, V), lambda i: (i, 0)),
            pl.BlockSpec((tm, 1), lambda i: (i, 0)),
            pl.BlockSpec((V, E), lambda i: (0, 0)),
            pl.BlockSpec((1, E), lambda i: (0, 0)),
            pl.BlockSpec((B, 1), lambda i: (0, 0)),
            pl.BlockSpec((1, 1), lambda i: (0, 0)),
            pl.BlockSpec((E, 2 * G), lambda i: (0, 0)),
            pl.BlockSpec((H, G), lambda i: (0, 0)),
            pl.BlockSpec((H, G), lambda i: (0, 0)),
            pl.BlockSpec((2 * H, 2 * G), lambda i: (0, 0)),
            pl.BlockSpec((H, G), lambda i: (0, 0)),
            pl.BlockSpec((H, G), lambda i: (0, 0)),
        ],
        out_specs=pl.BlockSpec((B, 2 * H), lambda i: (0, 0)),
        scratch_shapes=[
            pltpu.VMEM((L, B, E), jnp.float32),
            pltpu.VMEM((L * B, 2 * G), jnp.bfloat16),
            pltpu.VMEM((L * B, H), jnp.bfloat16),
            pltpu.VMEM((L * B, H), jnp.bfloat16),
        ],
        compiler_params=pltpu.CompilerParams(
            dimension_semantics=("arbitrary",)),
    )(text.reshape(N, V), len_row, embed_wt, embed_b, len_col, tmax,
      w0, whh0f, whh0b, w1, whh1f, whh1b)

    return out
